# baseline (speedup 1.0000x reference)
import functools

import jax
import jax.numpy as jnp
from jax import lax
from jax.experimental import pallas as pl
from jax.experimental.pallas import tpu as pltpu
from jax.experimental.pallas import tpu_sc as plsc

N_CLS = 1000
SEQ = 77
N_CTX = 4
CTX_DIM = 512
PROJ_DIM = 768

_NC = 2
_NS = 16
_NW = _NC * _NS

_CH = 56
_NCHK = 18
_NPOS = SEQ - N_CTX
_NITEM = _NPOS * _NCHK
_NCTX_ITEM = N_CTX * _NCHK
_MAXK = (_NITEM + _NW - 1) // _NW
_NB = 4
_NQUAD = (_MAXK + _NB - 1) // _NB

_LANE = 16
_NCHUNK16 = CTX_DIM // _LANE


def _sc_prompts_pm(table, ctx, tok_idx):
    mesh = plsc.VectorSubcoreMesh(core_axis_name="c", subcore_axis_name="s")

    @functools.partial(
        pl.kernel,
        out_type=jax.ShapeDtypeStruct((SEQ, N_CLS, CTX_DIM), jnp.float32),
        mesh=mesh,
        scratch_types=[
            pltpu.VMEM((N_CTX, CTX_DIM), jnp.float32),
            pltpu.VMEM((1, _CH), jnp.int32),
            pltpu.VMEM((1, _CH), jnp.int32),
            pltpu.VMEM((1, _CH), jnp.int32),
            pltpu.VMEM((1, _CH), jnp.int32),
            pltpu.VMEM((_CH, CTX_DIM), jnp.float32),
            pltpu.VMEM((_CH, CTX_DIM), jnp.float32),
            pltpu.VMEM((_CH, CTX_DIM), jnp.float32),
            pltpu.VMEM((_CH, CTX_DIM), jnp.float32),
            pltpu.SemaphoreType.DMA,
            pltpu.SemaphoreType.DMA,
            pltpu.SemaphoreType.DMA,
            pltpu.SemaphoreType.DMA,
            pltpu.SemaphoreType.DMA,
            pltpu.SemaphoreType.DMA,
            pltpu.SemaphoreType.DMA,
            pltpu.SemaphoreType.DMA,
        ],
        compiler_params=pltpu.CompilerParams(use_tc_tiling_on_sc=True),
    )
    def k(table_hbm, ctx_hbm, idx_hbm, out_hbm, ctx_v,
          ix0, ix1, ix2, ix3, buf0, buf1, buf2, buf3,
          s0, s1, s2, s3, t0, t1, t2, t3):
        wid = lax.axis_index("c") * _NS + lax.axis_index("s")
        count = (_NITEM - wid + _NW - 1) // _NW
        IX = (ix0, ix1, ix2, ix3)
        BUF = (buf0, buf1, buf2, buf3)
        SG = (s0, s1, s2, s3)
        SI = (t0, t1, t2, t3)

        def chunk_start(c):
            return jnp.where(c == _NCHK - 1, N_CLS - _CH, c * _CH)

        def item_meta(it):
            q = it // _NCHK
            p = jnp.where(q == 0, 0, q + N_CTX)
            return p, chunk_start(it % _NCHK)

        def stage_idx(k_, j):
            pltpu.async_copy(idx_hbm.at[wid + k_ * _NW], IX[j], SI[j])

        def wait_idx(k_, j):
            pltpu.make_async_copy(idx_hbm.at[wid + k_ * _NW],
                                  IX[j], SI[j]).wait()

        def issue_gather(j):
            pltpu.async_copy(table_hbm.at[IX[j].at[0]], BUF[j], SG[j])

        def wait_gather(j):
            pltpu.make_async_copy(table_hbm.at[IX[j].at[0]],
                                  BUF[j], SG[j]).wait()

        def issue_out(k_, j):
            p, c0 = item_meta(wid + k_ * _NW)
            pltpu.async_copy(BUF[j], out_hbm.at[p, pl.ds(c0, _CH)], SG[j])

        def wait_out(k_, j):
            p, c0 = item_meta(wid + k_ * _NW)
            pltpu.make_async_copy(BUF[j],
                                  out_hbm.at[p, pl.ds(c0, _CH)], SG[j]).wait()

        pltpu.sync_copy(ctx_hbm, ctx_v)
        for j in range(_NB):
            stage_idx(j, j)
        wait_idx(0, 0)
        issue_gather(0)
        wait_idx(1, 1)
        issue_gather(1)

        def ctx_item(t):
            p = 1 + t // _NCHK
            c0 = chunk_start(t % _NCHK)
            r_dyn = p - 1
            for r in range(N_CTX):
                @pl.when(r_dyn == r)
                def _():
                    vs = [ctx_v[r, pl.ds(_LANE * j, _LANE)]
                          for j in range(_NCHUNK16)]

                    def st(row, carry):
                        for j in range(_NCHUNK16):
                            buf3[row, pl.ds(_LANE * j, _LANE)] = vs[j]
                        return carry

                    lax.fori_loop(0, _CH, st, 0)
            pltpu.sync_copy(buf3, out_hbm.at[p, pl.ds(c0, _CH)])

        ctx_item(wid)
        ctx_item(_NW + wid)

        @pl.when(wid % 4 == 0)
        def _():
            ctx_item(2 * _NW + wid // 4)

        def slot(i, j):
            jn = (j + 2) % _NB

            @pl.when(i < count)
            def _():
                wait_gather(j)

                @pl.when(i + _NB < count)
                def _():
                    stage_idx(i + _NB, j)

                issue_out(i, j)

                @pl.when(i + 2 < count)
                def _():
                    @pl.when(i >= 2)
                    def _():
                        wait_out(i - 2, jn)

                    wait_idx(i + 2, jn)
                    issue_gather(jn)

        def quad(g, carry):
            i0 = _NB * g
            for j in range(_NB):
                slot(i0 + j, j)
            return carry

        lax.fori_loop(0, _NQUAD, quad, 0)

        @pl.when(count % _NB == 2)
        def _():
            for d in range(_NB):
                wait_out(count - _NB + d, (2 + d) % _NB)

        @pl.when(count % _NB == 1)
        def _():
            for d in range(_NB):
                wait_out(count - _NB + d, (1 + d) % _NB)

    return k(table, ctx, tok_idx)


def _tc_proj(ctx, W, b2):
    def body(ctx_ref, w_ref, b_ref, o_ref):
        o_ref[...] = (
            jnp.dot(ctx_ref[...], w_ref[...], preferred_element_type=jnp.float32)
            + b_ref[...]
        )

    return pl.pallas_call(
        body,
        out_shape=jax.ShapeDtypeStruct((N_CTX, PROJ_DIM), jnp.float32),
    )(ctx, W, b2)


def _build_tok_idx(tok):
    tok_t = tok.T
    tok_sel = jnp.concatenate([tok_t[:1], tok_t[N_CTX + 1:]], axis=0)
    starts = [min(c * _CH, N_CLS - _CH) for c in range(_NCHK)]
    chunks = jnp.stack([tok_sel[:, s:s + _CH] for s in starts], axis=1)
    return chunks.reshape(_NITEM, 1, _CH)


def kernel(ctx, table, W, b, tokenized_prompts):
    tok = tokenized_prompts.astype(jnp.int32)
    prompts_pm = _sc_prompts_pm(table, ctx, _build_tok_idx(tok))
    prompts = jnp.transpose(prompts_pm, (1, 0, 2))
    proj_ctx = _tc_proj(ctx, W, b.reshape(1, PROJ_DIM))
    return (tokenized_prompts, prompts, proj_ctx)


# speedup vs baseline: 1.0035x; 1.0035x over previous
import functools

import jax
import jax.numpy as jnp
from jax import lax
from jax.experimental import pallas as pl
from jax.experimental.pallas import tpu as pltpu
from jax.experimental.pallas import tpu_sc as plsc

N_CLS = 1000
SEQ = 77
N_CTX = 4
CTX_DIM = 512
PROJ_DIM = 768

_NC = 2
_NS = 16
_NW = _NC * _NS

_CH = 40
_NCHK = N_CLS // _CH
_NPOS = SEQ - N_CTX
_NITEM = _NPOS * _NCHK
_NCTX_ITEM = N_CTX * _NCHK
_MAXK = (_NITEM + _NW - 1) // _NW
_NB = 5
_GLEAD = 3
_NGRP = (_MAXK + _NB - 1) // _NB

_LANE = 16
_NCHUNK16 = CTX_DIM // _LANE


def _sc_prompts_pm(table, ctx, tok_idx):
    mesh = plsc.VectorSubcoreMesh(core_axis_name="c", subcore_axis_name="s")

    @functools.partial(
        pl.kernel,
        out_type=jax.ShapeDtypeStruct((SEQ, N_CLS, CTX_DIM), jnp.float32),
        mesh=mesh,
        scratch_types=(
            [pltpu.VMEM((N_CTX, CTX_DIM), jnp.float32)]
            + [pltpu.VMEM((1, _CH), jnp.int32)] * _NB
            + [pltpu.VMEM((_CH, CTX_DIM), jnp.float32)] * _NB
            + [pltpu.SemaphoreType.DMA] * (2 * _NB)
        ),
        compiler_params=pltpu.CompilerParams(use_tc_tiling_on_sc=True),
    )
    def k(table_hbm, ctx_hbm, idx_hbm, out_hbm, ctx_v, *rest):
        IX = rest[:_NB]
        BUF = rest[_NB:2 * _NB]
        SG = rest[2 * _NB:3 * _NB]
        SI = rest[3 * _NB:4 * _NB]
        wid = lax.axis_index("c") * _NS + lax.axis_index("s")
        count = (_NITEM - wid + _NW - 1) // _NW

        def chunk_start(c):
            return c * _CH

        def item_meta(it):
            q = it // _NCHK
            p = jnp.where(q == 0, 0, q + N_CTX)
            return p, chunk_start(it % _NCHK)

        def stage_idx(k_, j):
            pltpu.async_copy(idx_hbm.at[wid + k_ * _NW], IX[j], SI[j])

        def wait_idx(k_, j):
            pltpu.make_async_copy(idx_hbm.at[wid + k_ * _NW],
                                  IX[j], SI[j]).wait()

        def issue_gather(j):
            pltpu.async_copy(table_hbm.at[IX[j].at[0]], BUF[j], SG[j])

        def wait_gather(j):
            pltpu.make_async_copy(table_hbm.at[IX[j].at[0]],
                                  BUF[j], SG[j]).wait()

        def issue_out(k_, j):
            p, c0 = item_meta(wid + k_ * _NW)
            pltpu.async_copy(BUF[j], out_hbm.at[p, pl.ds(c0, _CH)], SG[j])

        def wait_out(k_, j):
            p, c0 = item_meta(wid + k_ * _NW)
            pltpu.make_async_copy(BUF[j],
                                  out_hbm.at[p, pl.ds(c0, _CH)], SG[j]).wait()

        pltpu.sync_copy(ctx_hbm, ctx_v)
        for j in range(_NB):
            stage_idx(j, j)
        for j in range(_GLEAD):
            wait_idx(j, j)
            issue_gather(j)

        def ctx_item(t):
            p = 1 + t // _NCHK
            c0 = chunk_start(t % _NCHK)
            fill = BUF[_NB - 1]
            r_dyn = p - 1
            for r in range(N_CTX):
                @pl.when(r_dyn == r)
                def _():
                    vs = [ctx_v[r, pl.ds(_LANE * j, _LANE)]
                          for j in range(_NCHUNK16)]

                    def st(row, carry):
                        for j in range(_NCHUNK16):
                            fill[row, pl.ds(_LANE * j, _LANE)] = vs[j]
                        return carry

                    lax.fori_loop(0, _CH, st, 0)
            pltpu.sync_copy(fill, out_hbm.at[p, pl.ds(c0, _CH)])

        ctx_item(wid)
        ctx_item(_NW + wid)
        ctx_item(2 * _NW + wid)

        @pl.when(wid % 8 == 0)
        def _():
            ctx_item(3 * _NW + wid // 8)

        def slot(i, j):
            jn = (j + _GLEAD) % _NB

            @pl.when(i < count)
            def _():
                wait_gather(j)

                @pl.when(i + _NB < count)
                def _():
                    stage_idx(i + _NB, j)

                issue_out(i, j)

                @pl.when(i + _GLEAD < count)
                def _():
                    @pl.when(i >= 2)
                    def _():
                        wait_out(i - 2, jn)

                    wait_idx(i + _GLEAD, jn)
                    issue_gather(jn)

        def grp(g, carry):
            i0 = _NB * g
            for j in range(_NB):
                slot(i0 + j, j)
            return carry

        lax.fori_loop(0, _NGRP, grp, 0)

        for cm in (57 % _NB, 58 % _NB):
            @pl.when(count % _NB == cm)
            def _():
                for d in range(_NB):
                    wait_out(count - _NB + d, (cm + d) % _NB)

    return k(table, ctx, tok_idx)


def _tc_proj(ctx, W, b2):
    def body(ctx_ref, w_ref, b_ref, o_ref):
        o_ref[...] = (
            jnp.dot(ctx_ref[...], w_ref[...], preferred_element_type=jnp.float32)
            + b_ref[...]
        )

    return pl.pallas_call(
        body,
        out_shape=jax.ShapeDtypeStruct((N_CTX, PROJ_DIM), jnp.float32),
    )(ctx, W, b2)


def _build_tok_idx(tok):
    tok_t = tok.T
    tok_sel = jnp.concatenate([tok_t[:1], tok_t[N_CTX + 1:]], axis=0)
    chunks = jnp.stack(
        [tok_sel[:, c * _CH:(c + 1) * _CH] for c in range(_NCHK)], axis=1)
    return chunks.reshape(_NITEM, 1, _CH)


def kernel(ctx, table, W, b, tokenized_prompts):
    tok = tokenized_prompts.astype(jnp.int32)
    prompts_pm = _sc_prompts_pm(table, ctx, _build_tok_idx(tok))
    prompts = jnp.transpose(prompts_pm, (1, 0, 2))
    proj_ctx = _tc_proj(ctx, W, b.reshape(1, PROJ_DIM))
    return (tokenized_prompts, prompts, proj_ctx)
